# baseline (speedup 1.0000x reference)
import functools

import jax
import jax.numpy as jnp
from jax import lax
from jax.experimental import pallas as pl
from jax.experimental.pallas import tpu as pltpu
from jax.experimental.pallas import tpu_sc as plsc

NC = 2
NS = 16
NW = NC * NS
CHUNK = 128
NBUF = 12


@functools.lru_cache(maxsize=None)
def _make_gather(n_chunks, v, d):
    mesh = plsc.VectorSubcoreMesh(core_axis_name="c", subcore_axis_name="s")
    n_rows = n_chunks * CHUNK

    @functools.partial(
        pl.kernel,
        out_type=jax.ShapeDtypeStruct((NW * n_rows, d), jnp.float32),
        mesh=mesh,
        scratch_types=[
            pltpu.VMEM((n_chunks, CHUNK), jnp.int32),
            pltpu.VMEM((NBUF, CHUNK, d), jnp.float32),
            pltpu.SemaphoreType.DMA((NBUF,)),
            pltpu.SemaphoreType.DMA((NBUF,)),
        ],
        compiler_params=pltpu.CompilerParams(use_tc_tiling_on_sc=False),
    )
    def gather_kernel(idx_hbm, table_hbm, out_hbm, idx_v, rows_v, gsem, osem):
        wid = lax.axis_index("s") * NC + lax.axis_index("c")
        base = wid * n_rows
        pltpu.sync_copy(idx_hbm.at[wid], idx_v)

        gathers = [None] * n_chunks
        outs = [None] * n_chunks

        def start_gather(j):
            gathers[j] = pltpu.async_copy(
                table_hbm.at[idx_v.at[j]], rows_v.at[j % NBUF],
                gsem.at[j % NBUF])

        for j in range(min(NBUF, n_chunks)):
            start_gather(j)
        for j in range(n_chunks):
            b = j % NBUF
            gathers[j].wait()
            outs[j] = pltpu.async_copy(
                rows_v.at[b], out_hbm.at[pl.ds(base + j * CHUNK, CHUNK)],
                osem.at[b])
            if j + NBUF < n_chunks:
                outs[j].wait()
                start_gather(j + NBUF)
        for j in range(max(0, n_chunks - NBUF), n_chunks):
            outs[j].wait()

    return gather_kernel


def kernel(sequence, table):
    b, s = sequence.shape
    v, d = table.shape
    flat = sequence.reshape(-1).astype(jnp.int32)
    n = flat.shape[0]
    per_w = -(-n // (NW * CHUNK)) * CHUNK
    n_pad = NW * per_w
    if n_pad != n:
        flat = jnp.pad(flat, (0, n_pad - n))
    idx3 = flat.reshape(NW, per_w // CHUNK, CHUNK)
    out = _make_gather(per_w // CHUNK, v, d)(idx3, table)
    return out[:n].reshape(b, s, d)
